# Initial kernel scaffold; baseline (speedup 1.0000x reference)
import jax
import jax.numpy as jnp
from jax.experimental import pallas as pl


def kernel(ligand_pos, ligand_v, edge_index, time_step, W_node, W_time, W_msg1, b_msg1, W_msg2, W_upd, W_coord, W_out):
    raise NotImplementedError("write your pallas kernel here")


# speedup vs baseline: 3.4307x; 3.4307x over previous
import functools

import numpy as np
import jax
import jax.numpy as jnp
from jax import lax
from jax.experimental import pallas as pl
from jax.experimental.pallas import tpu as pltpu
from jax.experimental.pallas import tpu_sc as plsc

_NC = 2
_NS = 16
_NW = _NC * _NS

_BE = 80
_BN = 2000
_BET = 4000


def _sc_mesh():
    return plsc.VectorSubcoreMesh(
        core_axis_name="c", subcore_axis_name="s", num_cores=_NC, num_subcores=_NS
    )


def _embed_body(v_ref, t_ref, pos_ref, freq_ref, wnode_ref, wtime_ref, h_ref, x_ref):
    k = wnode_ref.shape[0]
    bn = v_ref.shape[0]
    hdim = wnode_ref.shape[1]
    onehot = (v_ref[...] == lax.broadcasted_iota(jnp.int32, (bn, k), 1)).astype(
        jnp.float32
    )
    arg = t_ref[...].astype(jnp.float32) * freq_ref[...]
    emb = jnp.concatenate([jnp.sin(arg), jnp.cos(arg)], axis=1)
    h_ref[...] = jnp.dot(
        onehot, wnode_ref[...], preferred_element_type=jnp.float32
    ) + jnp.dot(emb, wtime_ref[...], preferred_element_type=jnp.float32)
    pos = pos_ref[...]
    x_ref[...] = jnp.concatenate(
        [pos, jnp.zeros((bn, hdim - 3), jnp.float32)], axis=1
    )


def _embed(v2, t2, pos, freq, w_node, w_time, n, h):
    grid = (n // _BN,)
    return pl.pallas_call(
        _embed_body,
        grid=grid,
        in_specs=[
            pl.BlockSpec((_BN, 1), lambda i: (i, 0)),
            pl.BlockSpec((_BN, 1), lambda i: (i, 0)),
            pl.BlockSpec((_BN, 3), lambda i: (i, 0)),
            pl.BlockSpec((1, h // 2), lambda i: (0, 0)),
            pl.BlockSpec(w_node.shape, lambda i: (0, 0)),
            pl.BlockSpec((h, h), lambda i: (0, 0)),
        ],
        out_specs=[
            pl.BlockSpec((_BN, h), lambda i: (i, 0)),
            pl.BlockSpec((_BN, h), lambda i: (i, 0)),
        ],
        out_shape=[
            jax.ShapeDtypeStruct((n, h), jnp.float32),
            jax.ShapeDtypeStruct((n, h), jnp.float32),
        ],
    )(v2, t2, pos, freq, w_node, w_time)


def _proj_body(h_ref, w1a_ref, w1b_ref, t1_ref, t2_ref):
    hb = h_ref[...]
    t1_ref[...] = jnp.dot(hb, w1a_ref[...], preferred_element_type=jnp.float32)
    t2_ref[...] = jnp.dot(hb, w1b_ref[...], preferred_element_type=jnp.float32)


def _proj(hmat, w1a, w1b, n, h):
    grid = (n // _BN,)
    return pl.pallas_call(
        _proj_body,
        grid=grid,
        in_specs=[
            pl.BlockSpec((_BN, h), lambda i: (i, 0)),
            pl.BlockSpec((h, h), lambda i: (0, 0)),
            pl.BlockSpec((h, h), lambda i: (0, 0)),
        ],
        out_specs=[
            pl.BlockSpec((_BN, h), lambda i: (i, 0)),
            pl.BlockSpec((_BN, h), lambda i: (i, 0)),
        ],
        out_shape=[
            jax.ShapeDtypeStruct((n, h), jnp.float32),
            jax.ShapeDtypeStruct((n, h), jnp.float32),
        ],
    )(hmat, w1a, w1b)


def _edge_body(g1_ref, g2_ref, gx1_ref, gx2_ref, w1d_ref, b_ref, w2_ref, wc_ref,
               m_ref, dx_ref):
    rel = gx2_ref[...] - gx1_ref[...]
    dist = jnp.sqrt(jnp.sum(rel * rel, axis=1, keepdims=True) + 1e-8)
    a = g1_ref[...] + g2_ref[...] + dist * w1d_ref[...] + b_ref[...]
    m1 = a * jax.nn.sigmoid(a)
    mm = jnp.dot(m1, w2_ref[...], preferred_element_type=jnp.float32)
    m = mm * jax.nn.sigmoid(mm)
    m_ref[...] = m
    coef = jnp.tanh(jnp.sum(m * wc_ref[...], axis=1, keepdims=True))
    dx_ref[...] = coef * rel / (dist + 1.0)


def _edge(g1, g2, gx1, gx2, w1d, b, w2, wc, e, h):
    grid = (e // _BET,)
    return pl.pallas_call(
        _edge_body,
        grid=grid,
        in_specs=[
            pl.BlockSpec((_BET, h), lambda i: (i, 0)),
            pl.BlockSpec((_BET, h), lambda i: (i, 0)),
            pl.BlockSpec((_BET, h), lambda i: (i, 0)),
            pl.BlockSpec((_BET, h), lambda i: (i, 0)),
            pl.BlockSpec((1, h), lambda i: (0, 0)),
            pl.BlockSpec((1, h), lambda i: (0, 0)),
            pl.BlockSpec((h, h), lambda i: (0, 0)),
            pl.BlockSpec((1, h), lambda i: (0, 0)),
        ],
        out_specs=[
            pl.BlockSpec((_BET, h), lambda i: (i, 0)),
            pl.BlockSpec((_BET, h), lambda i: (i, 0)),
        ],
        out_shape=[
            jax.ShapeDtypeStruct((e, h), jnp.float32),
            jax.ShapeDtypeStruct((e, h), jnp.float32),
        ],
    )(g1, g2, gx1, gx2, w1d, b, w2, wc)


def _update_body(h_ref, x_ref, agg_ref, dx_ref, wu_ref, ho_ref, xo_ref):
    u = jnp.dot(agg_ref[...], wu_ref[...], preferred_element_type=jnp.float32)
    ho_ref[...] = h_ref[...] + u * jax.nn.sigmoid(u)
    xo_ref[...] = x_ref[...] + dx_ref[...]


def _update(hmat, x128, agg, dx, wu, n, h):
    grid = (n // _BN,)
    return pl.pallas_call(
        _update_body,
        grid=grid,
        in_specs=[
            pl.BlockSpec((_BN, h), lambda i: (i, 0)),
            pl.BlockSpec((_BN, h), lambda i: (i, 0)),
            pl.BlockSpec((_BN, h), lambda i: (i, 0)),
            pl.BlockSpec((_BN, h), lambda i: (i, 0)),
            pl.BlockSpec((h, h), lambda i: (0, 0)),
        ],
        out_specs=[
            pl.BlockSpec((_BN, h), lambda i: (i, 0)),
            pl.BlockSpec((_BN, h), lambda i: (i, 0)),
        ],
        out_shape=[
            jax.ShapeDtypeStruct((n, h), jnp.float32),
            jax.ShapeDtypeStruct((n, h), jnp.float32),
        ],
    )(hmat, x128, agg, dx, wu)


def _final_body(h_ref, x_ref, pos_ref, wout_ref, o_ref):
    eps = x_ref[..., :3] - pos_ref[...]
    logits = jnp.dot(h_ref[...], wout_ref[...], preferred_element_type=jnp.float32)
    o_ref[...] = jnp.concatenate([eps, logits], axis=1)


def _final(hmat, x128, pos, wout, n, h, k):
    grid = (n // _BN,)
    return pl.pallas_call(
        _final_body,
        grid=grid,
        in_specs=[
            pl.BlockSpec((_BN, h), lambda i: (i, 0)),
            pl.BlockSpec((_BN, h), lambda i: (i, 0)),
            pl.BlockSpec((_BN, 3), lambda i: (i, 0)),
            pl.BlockSpec((h, k), lambda i: (0, 0)),
        ],
        out_specs=pl.BlockSpec((_BN, 3 + k), lambda i: (i, 0)),
        out_shape=jax.ShapeDtypeStruct((n, 3 + k), jnp.float32),
    )(hmat, x128, pos, wout)


def _make_gather(n, e, h):
    epw = e // _NW
    nblk = epw // _BE
    mesh = _sc_mesh()

    @functools.partial(
        pl.kernel,
        out_type=[
            jax.ShapeDtypeStruct((e, h), jnp.float32),
            jax.ShapeDtypeStruct((e, h), jnp.float32),
            jax.ShapeDtypeStruct((e, h), jnp.float32),
            jax.ShapeDtypeStruct((e, h), jnp.float32),
        ],
        mesh=mesh,
        scratch_types=[
            pltpu.VMEM((nblk, _BE), jnp.int32),
            pltpu.VMEM((nblk, _BE), jnp.int32),
            pltpu.VMEM((_BE, h), jnp.float32),
            pltpu.VMEM((_BE, h), jnp.float32),
            pltpu.VMEM((_BE, h), jnp.float32),
            pltpu.VMEM((_BE, h), jnp.float32),
            pltpu.SemaphoreType.DMA,
            pltpu.SemaphoreType.DMA,
            pltpu.SemaphoreType.DMA,
            pltpu.SemaphoreType.DMA,
        ],
    )
    def gather(t1, t2, x128, src3, dst3, g1, g2, gx1, gx2,
               idxs, idxd, b1, b2, bx1, bx2, s0, s1, s2, s3):
        c = lax.axis_index("c")
        s = lax.axis_index("s")
        wid = s * _NC + c
        ebase = wid * epw
        pltpu.sync_copy(src3.at[wid], idxs)
        pltpu.sync_copy(dst3.at[wid], idxd)

        def step(i, carry):
            off = ebase + i * _BE
            c1 = pltpu.async_copy(t1.at[idxs.at[i]], b1, s0)
            c2 = pltpu.async_copy(t2.at[idxd.at[i]], b2, s1)
            c3 = pltpu.async_copy(x128.at[idxs.at[i]], bx1, s2)
            c4 = pltpu.async_copy(x128.at[idxd.at[i]], bx2, s3)
            c1.wait()
            c2.wait()
            c3.wait()
            c4.wait()
            w1 = pltpu.async_copy(b1, g1.at[pl.ds(off, _BE)], s0)
            w2 = pltpu.async_copy(b2, g2.at[pl.ds(off, _BE)], s1)
            w3 = pltpu.async_copy(bx1, gx1.at[pl.ds(off, _BE)], s2)
            w4 = pltpu.async_copy(bx2, gx2.at[pl.ds(off, _BE)], s3)
            w1.wait()
            w2.wait()
            w3.wait()
            w4.wait()
            return carry

        lax.fori_loop(0, nblk, step, 0)

    return gather


def _make_scatter(n, e, h):
    ept = e // _NS
    nblk = ept // _BE
    tiles_out = 10
    rows_per_tile = n // tiles_out
    mesh = _sc_mesh()

    @functools.partial(
        pl.kernel,
        out_type=[
            jax.ShapeDtypeStruct((n, h), jnp.float32),
            jax.ShapeDtypeStruct((n, h), jnp.float32),
        ],
        mesh=mesh,
        scratch_types=[
            pltpu.VMEM((nblk, _BE), jnp.int32),
            pltpu.VMEM((_BE, h), jnp.float32),
            pltpu.VMEM_SHARED((n, h), jnp.float32),
            pltpu.SemaphoreType.DMA,
        ],
    )
    def scatter(mh, mx, dst3, zeros_nh, agg, dxs,
                idxd, bm, sp, s0):
        c = lax.axis_index("c")
        s = lax.axis_index("s")

        @pl.when(s < tiles_out)
        def _zero():
            r0 = s * rows_per_tile
            pltpu.sync_copy(zeros_nh.at[pl.ds(r0, rows_per_tile)],
                            sp.at[pl.ds(r0, rows_per_tile)])

        plsc.subcore_barrier()

        pltpu.sync_copy(dst3.at[s], idxd)

        def step_m(i, carry):
            off = s * ept + i * _BE
            pltpu.async_copy(mh.at[pl.ds(off, _BE)], bm, s0).wait()
            pltpu.sync_copy(bm, sp.at[idxd.at[i]], add=True)
            return carry

        def step_x(i, carry):
            off = s * ept + i * _BE
            pltpu.async_copy(mx.at[pl.ds(off, _BE)], bm, s0).wait()
            pltpu.sync_copy(bm, sp.at[idxd.at[i]], add=True)
            return carry

        @pl.when(c == 0)
        def _loop_m():
            lax.fori_loop(0, nblk, step_m, 0)

        @pl.when(c == 1)
        def _loop_x():
            lax.fori_loop(0, nblk, step_x, 0)

        plsc.subcore_barrier()

        @pl.when(jnp.logical_and(s < tiles_out, c == 0))
        def _out_m():
            r0 = s * rows_per_tile
            pltpu.sync_copy(sp.at[pl.ds(r0, rows_per_tile)],
                            agg.at[pl.ds(r0, rows_per_tile)])

        @pl.when(jnp.logical_and(s < tiles_out, c == 1))
        def _out_x():
            r0 = s * rows_per_tile
            pltpu.sync_copy(sp.at[pl.ds(r0, rows_per_tile)],
                            dxs.at[pl.ds(r0, rows_per_tile)])

    return scatter


def kernel(ligand_pos, ligand_v, edge_index, time_step, W_node, W_time,
           W_msg1, b_msg1, W_msg2, W_upd, W_coord, W_out):
    n = ligand_pos.shape[0]
    e = edge_index.shape[1]
    k = W_node.shape[0]
    h = W_node.shape[1]
    n_layers = W_msg1.shape[0]

    nblk_g = e // _NW // _BE
    nblk_s = e // _NS // _BE
    src3 = edge_index[0].astype(jnp.int32).reshape(_NW, nblk_g, _BE)
    dst3 = edge_index[1].astype(jnp.int32).reshape(_NW, nblk_g, _BE)
    dst3s = edge_index[1].astype(jnp.int32).reshape(_NS, nblk_s, _BE)
    v2 = ligand_v.astype(jnp.int32).reshape(n, 1)
    t2 = time_step.astype(jnp.int32).reshape(n, 1)
    half = h // 2
    freq = jnp.asarray(
        np.exp(np.arange(half, dtype=np.float32) * -(np.log(10000.0) / (half - 1)))
    ).reshape(1, half)
    zeros_nh = jnp.zeros((n, h), jnp.float32)

    gather = _make_gather(n, e, h)
    scatter = _make_scatter(n, e, h)

    hmat, x128 = _embed(v2, t2, ligand_pos, freq, W_node, W_time, n, h)

    for l in range(n_layers):
        w1a = W_msg1[l, :h, :]
        w1b = W_msg1[l, h:2 * h, :]
        w1d = W_msg1[l, 2 * h:2 * h + 1, :]
        bias = b_msg1[l].reshape(1, h)
        wc = W_coord[l, :, 0].reshape(1, h)

        t1h, t2h = _proj(hmat, w1a, w1b, n, h)
        g1, g2, gx1, gx2 = gather(t1h, t2h, x128, src3, dst3)
        mh, mx = _edge(g1, g2, gx1, gx2, w1d, bias, W_msg2[l], wc, e, h)
        agg, dxs = scatter(mh, mx, dst3s, zeros_nh)
        hmat, x128 = _update(hmat, x128, agg, dxs, W_upd[l], n, h)

    return _final(hmat, x128, ligand_pos, W_out, n, h, k)


# speedup vs baseline: 3.9029x; 1.1376x over previous
import functools

import numpy as np
import jax
import jax.numpy as jnp
from jax import lax
from jax.experimental import pallas as pl
from jax.experimental.pallas import tpu as pltpu
from jax.experimental.pallas import tpu_sc as plsc

_NC = 2
_NS = 16
_NW = _NC * _NS

_BE = 80
_BN = 2000
_BET = 4000


def _sc_mesh():
    return plsc.VectorSubcoreMesh(
        core_axis_name="c", subcore_axis_name="s", num_cores=_NC, num_subcores=_NS
    )


def _embed_body(v_ref, t_ref, pos_ref, freq_ref, wnode_ref, wtime_ref, h_ref, x_ref):
    k = wnode_ref.shape[0]
    bn = v_ref.shape[0]
    hdim = wnode_ref.shape[1]
    onehot = (v_ref[...] == lax.broadcasted_iota(jnp.int32, (bn, k), 1)).astype(
        jnp.float32
    )
    arg = t_ref[...].astype(jnp.float32) * freq_ref[...]
    emb = jnp.concatenate([jnp.sin(arg), jnp.cos(arg)], axis=1)
    h_ref[...] = jnp.dot(
        onehot, wnode_ref[...], preferred_element_type=jnp.float32
    ) + jnp.dot(emb, wtime_ref[...], preferred_element_type=jnp.float32)
    pos = pos_ref[...]
    x_ref[...] = jnp.concatenate(
        [pos, jnp.zeros((bn, hdim - 3), jnp.float32)], axis=1
    )


def _embed(v2, t2, pos, freq, w_node, w_time, n, h):
    grid = (n // _BN,)
    return pl.pallas_call(
        _embed_body,
        grid=grid,
        in_specs=[
            pl.BlockSpec((_BN, 1), lambda i: (i, 0)),
            pl.BlockSpec((_BN, 1), lambda i: (i, 0)),
            pl.BlockSpec((_BN, 3), lambda i: (i, 0)),
            pl.BlockSpec((1, h // 2), lambda i: (0, 0)),
            pl.BlockSpec(w_node.shape, lambda i: (0, 0)),
            pl.BlockSpec((h, h), lambda i: (0, 0)),
        ],
        out_specs=[
            pl.BlockSpec((_BN, h), lambda i: (i, 0)),
            pl.BlockSpec((_BN, h), lambda i: (i, 0)),
        ],
        out_shape=[
            jax.ShapeDtypeStruct((n, h), jnp.float32),
            jax.ShapeDtypeStruct((n, h), jnp.float32),
        ],
    )(v2, t2, pos, freq, w_node, w_time)


def _proj_body(h_ref, w1a_ref, w1b_ref, t1_ref, t2_ref):
    hb = h_ref[...]
    t1_ref[...] = jnp.dot(hb, w1a_ref[...], preferred_element_type=jnp.float32)
    t2_ref[...] = jnp.dot(hb, w1b_ref[...], preferred_element_type=jnp.float32)


def _proj(hmat, w1a, w1b, n, h):
    grid = (n // _BN,)
    return pl.pallas_call(
        _proj_body,
        grid=grid,
        in_specs=[
            pl.BlockSpec((_BN, h), lambda i: (i, 0)),
            pl.BlockSpec((h, h), lambda i: (0, 0)),
            pl.BlockSpec((h, h), lambda i: (0, 0)),
        ],
        out_specs=[
            pl.BlockSpec((_BN, h), lambda i: (i, 0)),
            pl.BlockSpec((_BN, h), lambda i: (i, 0)),
        ],
        out_shape=[
            jax.ShapeDtypeStruct((n, h), jnp.float32),
            jax.ShapeDtypeStruct((n, h), jnp.float32),
        ],
    )(hmat, w1a, w1b)


def _edge_body(g_ref, gx_ref, w1d_ref, b_ref, w2_ref, wc_ref, m_ref, mx_ref):
    rel = gx_ref[...]
    dist = jnp.sqrt(jnp.sum(rel * rel, axis=1, keepdims=True) + 1e-8)
    a = g_ref[...] + dist * w1d_ref[...] + b_ref[...]
    m1 = a * jax.nn.sigmoid(a)
    mm = jnp.dot(m1, w2_ref[...], preferred_element_type=jnp.float32)
    m = mm * jax.nn.sigmoid(mm)
    m_ref[...] = m
    coef = jnp.tanh(jnp.sum(m * wc_ref[...], axis=1, keepdims=True))
    mx_ref[...] = coef * rel / (dist + 1.0)


def _edge(g, gxr, w1d, b, w2, wc, e, h):
    grid = (e // _BET,)
    return pl.pallas_call(
        _edge_body,
        grid=grid,
        in_specs=[
            pl.BlockSpec((_BET, h), lambda i: (i, 0)),
            pl.BlockSpec((_BET, 16), lambda i: (i, 0)),
            pl.BlockSpec((1, h), lambda i: (0, 0)),
            pl.BlockSpec((1, h), lambda i: (0, 0)),
            pl.BlockSpec((h, h), lambda i: (0, 0)),
            pl.BlockSpec((1, h), lambda i: (0, 0)),
        ],
        out_specs=[
            pl.BlockSpec((_BET, h), lambda i: (i, 0)),
            pl.BlockSpec((_BET, 16), lambda i: (i, 0)),
        ],
        out_shape=[
            jax.ShapeDtypeStruct((e, h), jnp.float32),
            jax.ShapeDtypeStruct((e, 16), jnp.float32),
        ],
    )(g, gxr, w1d, b, w2, wc)


def _update_body(h_ref, x_ref, agg_ref, dx_ref, wu_ref, ho_ref, xo_ref):
    u = jnp.dot(agg_ref[...], wu_ref[...], preferred_element_type=jnp.float32)
    ho_ref[...] = h_ref[...] + u * jax.nn.sigmoid(u)
    xo_ref[...] = x_ref[...] + dx_ref[...]


def _update(hmat, x128, agg, dx, wu, n, h):
    grid = (n // _BN,)
    return pl.pallas_call(
        _update_body,
        grid=grid,
        in_specs=[
            pl.BlockSpec((_BN, h), lambda i: (i, 0)),
            pl.BlockSpec((_BN, h), lambda i: (i, 0)),
            pl.BlockSpec((_BN, h), lambda i: (i, 0)),
            pl.BlockSpec((_BN, h), lambda i: (i, 0)),
            pl.BlockSpec((h, h), lambda i: (0, 0)),
        ],
        out_specs=[
            pl.BlockSpec((_BN, h), lambda i: (i, 0)),
            pl.BlockSpec((_BN, h), lambda i: (i, 0)),
        ],
        out_shape=[
            jax.ShapeDtypeStruct((n, h), jnp.float32),
            jax.ShapeDtypeStruct((n, h), jnp.float32),
        ],
    )(hmat, x128, agg, dx, wu)


def _final_body(h_ref, x_ref, pos_ref, wout_ref, o_ref):
    eps = x_ref[..., :3] - pos_ref[...]
    logits = jnp.dot(h_ref[...], wout_ref[...], preferred_element_type=jnp.float32)
    o_ref[...] = jnp.concatenate([eps, logits], axis=1)


def _final(hmat, x128, pos, wout, n, h, k):
    grid = (n // _BN,)
    return pl.pallas_call(
        _final_body,
        grid=grid,
        in_specs=[
            pl.BlockSpec((_BN, h), lambda i: (i, 0)),
            pl.BlockSpec((_BN, h), lambda i: (i, 0)),
            pl.BlockSpec((_BN, 3), lambda i: (i, 0)),
            pl.BlockSpec((h, k), lambda i: (0, 0)),
        ],
        out_specs=pl.BlockSpec((_BN, 3 + k), lambda i: (i, 0)),
        out_shape=jax.ShapeDtypeStruct((n, 3 + k), jnp.float32),
    )(hmat, x128, pos, wout)


def _make_gather(n, e, h):
    epw = e // _NW
    nblk = epw // _BE
    mesh = _sc_mesh()

    npair = nblk // 2

    @functools.partial(
        pl.kernel,
        out_type=[
            jax.ShapeDtypeStruct((e, h), jnp.float32),
            jax.ShapeDtypeStruct((e, 16), jnp.float32),
        ],
        mesh=mesh,
        scratch_types=[
            pltpu.VMEM((nblk, _BE), jnp.int32),
            pltpu.VMEM((nblk, _BE), jnp.int32),
            pltpu.VMEM((_BE, h), jnp.float32),
            pltpu.VMEM((_BE, h), jnp.float32),
            pltpu.VMEM((_BE, h), jnp.float32),
            pltpu.VMEM((_BE, h), jnp.float32),
            pltpu.VMEM((_BE, h), jnp.float32),
            pltpu.VMEM((_BE, h), jnp.float32),
            pltpu.VMEM((_BE, 16), jnp.float32),
            pltpu.VMEM((_BE, 16), jnp.float32),
            pltpu.SemaphoreType.DMA,
            pltpu.SemaphoreType.DMA,
            pltpu.SemaphoreType.DMA,
            pltpu.SemaphoreType.DMA,
            pltpu.SemaphoreType.DMA,
        ],
    )
    def gather(t1, t2, x128, src3, dst3, gsum, grel,
               idxs, idxd,
               b1a, b2a, b1b, b2b, bx1, bx2, crela, crelb,
               ga, gb, gx, wa, wb):
        c = lax.axis_index("c")
        s = lax.axis_index("s")
        wid = s * _NC + c
        ebase = wid * epw
        pltpu.sync_copy(src3.at[wid], idxs)
        pltpu.sync_copy(dst3.at[wid], idxd)

        def issue_h(i, b1, b2, sem):
            d1 = pltpu.async_copy(t1.at[idxs.at[i]], b1, sem)
            d2 = pltpu.async_copy(t2.at[idxd.at[i]], b2, sem)
            return (d1, d2)

        def issue_x(i):
            d3 = pltpu.async_copy(x128.at[idxs.at[i]], bx1, gx)
            d4 = pltpu.async_copy(x128.at[idxd.at[i]], bx2, gx)
            return (d3, d4)

        def add_h(b1, b2):
            def body(rr, carry):
                r = rr * 8
                for kk in range(8):
                    for cc in range(8):
                        sl = (r + kk, pl.ds(cc * 16, 16))
                        b1[sl] = b1[sl] + b2[sl]
                return carry

            lax.fori_loop(0, _BE // 8, body, 0)

        def sub_x(crel):
            def body(rr, carry):
                r = rr * 8
                for kk in range(8):
                    crel[r + kk] = (bx2[r + kk, pl.ds(0, 16)]
                                    - bx1[r + kk, pl.ds(0, 16)])
                return carry

            lax.fori_loop(0, _BE // 8, body, 0)

        def writeback(i, b1, crel, sem):
            off = ebase + i * _BE
            d1 = pltpu.async_copy(b1, gsum.at[pl.ds(off, _BE)], sem)
            d2 = pltpu.async_copy(crel, grel.at[pl.ds(off, _BE)], sem)
            return (d1, d2)

        def pair(j, carry):
            i0 = 2 * j
            i1 = 2 * j + 1
            da = issue_h(i0, b1a, b2a, ga)
            dxa = issue_x(i0)
            db = issue_h(i1, b1b, b2b, gb)
            for d in da:
                d.wait()
            for d in dxa:
                d.wait()
            add_h(b1a, b2a)
            sub_x(crela)
            dxb = issue_x(i1)
            wda = writeback(i0, b1a, crela, wa)
            for d in db:
                d.wait()
            for d in dxb:
                d.wait()
            add_h(b1b, b2b)
            sub_x(crelb)
            wdb = writeback(i1, b1b, crelb, wb)
            for d in wda:
                d.wait()
            for d in wdb:
                d.wait()
            return carry

        lax.fori_loop(0, npair, pair, 0)
        if nblk % 2:
            i0 = nblk - 1
            da = issue_h(i0, b1a, b2a, ga)
            dxa = issue_x(i0)
            for d in da:
                d.wait()
            for d in dxa:
                d.wait()
            add_h(b1a, b2a)
            sub_x(crela)
            wda = writeback(i0, b1a, crela, wa)
            for d in wda:
                d.wait()

    return gather


def _make_scatter(n, e, h):
    ept = e // _NS
    nblk = ept // _BE
    tiles_out = 10
    rows_per_tile = n // tiles_out
    mesh = _sc_mesh()

    pblk = nblk // 2
    ppair = pblk // 2

    @functools.partial(
        pl.kernel,
        out_type=[
            jax.ShapeDtypeStruct((n, h), jnp.float32),
            jax.ShapeDtypeStruct((n, h), jnp.float32),
        ],
        mesh=mesh,
        scratch_types=[
            pltpu.VMEM((pblk, _BE), jnp.int32),
            pltpu.VMEM((_BE, h), jnp.float32),
            pltpu.VMEM((_BE, h), jnp.float32),
            pltpu.VMEM((_BE, 16), jnp.float32),
            pltpu.VMEM_SHARED((n, h), jnp.float32),
            pltpu.SemaphoreType.DMA,
            pltpu.SemaphoreType.DMA,
        ],
    )
    def scatter(mh, mx, dst3, zeros_nh, agg, dxs,
                idxd, bma, bmb, bxa, sp, sa, sb):
        c = lax.axis_index("c")
        s = lax.axis_index("s")

        @pl.when(s < tiles_out)
        def _zero():
            r0 = s * rows_per_tile
            pltpu.sync_copy(zeros_nh.at[pl.ds(r0, rows_per_tile)],
                            sp.at[pl.ds(r0, rows_per_tile)])

        plsc.subcore_barrier()

        @pl.when(c == 0)
        def _core_m():
            for p in range(2):
                pltpu.sync_copy(dst3.at[s, p], idxd)
                base = p * pblk

                def pair(j, carry):
                    i0 = 2 * j
                    i1 = 2 * j + 1
                    da = pltpu.async_copy(
                        mh.at[pl.ds(s * ept + (base + i0) * _BE, _BE)], bma, sa)
                    db = pltpu.async_copy(
                        mh.at[pl.ds(s * ept + (base + i1) * _BE, _BE)], bmb, sb)
                    da.wait()
                    pltpu.sync_copy(bma, sp.at[idxd.at[i0]], add=True)
                    db.wait()
                    pltpu.sync_copy(bmb, sp.at[idxd.at[i1]], add=True)
                    return carry

                lax.fori_loop(0, ppair, pair, 0)
                if pblk % 2:
                    i0 = pblk - 1
                    pltpu.async_copy(
                        mh.at[pl.ds(s * ept + (base + i0) * _BE, _BE)],
                        bma, sa).wait()
                    pltpu.sync_copy(bma, sp.at[idxd.at[i0]], add=True)

        @pl.when(c == 1)
        def _core_x():
            def zbody(rr, carry):
                r = rr * 8
                for kk in range(8):
                    for cc in range(8):
                        bma[r + kk, pl.ds(cc * 16, 16)] = jnp.zeros(
                            (16,), jnp.float32)
                return carry

            lax.fori_loop(0, _BE // 8, zbody, 0)

            def expand(bx):
                def body(rr, carry):
                    r = rr * 8
                    for kk in range(8):
                        bma[r + kk, pl.ds(0, 16)] = bx[r + kk]
                    return carry

                lax.fori_loop(0, _BE // 8, body, 0)

            for p in range(2):
                pltpu.sync_copy(dst3.at[s, p], idxd)
                base = p * pblk

                def pair(j, carry):
                    i0 = 2 * j
                    i1 = 2 * j + 1
                    da = pltpu.async_copy(
                        mx.at[pl.ds(s * ept + (base + i0) * _BE, _BE)], bxa, sa)
                    da.wait()
                    expand(bxa)
                    db = pltpu.async_copy(
                        mx.at[pl.ds(s * ept + (base + i1) * _BE, _BE)], bxa, sb)
                    pltpu.sync_copy(bma, sp.at[idxd.at[i0]], add=True)
                    db.wait()
                    expand(bxa)
                    pltpu.sync_copy(bma, sp.at[idxd.at[i1]], add=True)
                    return carry

                lax.fori_loop(0, ppair, pair, 0)
                if pblk % 2:
                    i0 = pblk - 1
                    pltpu.async_copy(
                        mx.at[pl.ds(s * ept + (base + i0) * _BE, _BE)],
                        bxa, sa).wait()
                    expand(bxa)
                    pltpu.sync_copy(bma, sp.at[idxd.at[i0]], add=True)

        plsc.subcore_barrier()

        @pl.when(jnp.logical_and(s < tiles_out, c == 0))
        def _out_m():
            r0 = s * rows_per_tile
            pltpu.sync_copy(sp.at[pl.ds(r0, rows_per_tile)],
                            agg.at[pl.ds(r0, rows_per_tile)])

        @pl.when(jnp.logical_and(s < tiles_out, c == 1))
        def _out_x():
            r0 = s * rows_per_tile
            pltpu.sync_copy(sp.at[pl.ds(r0, rows_per_tile)],
                            dxs.at[pl.ds(r0, rows_per_tile)])

    return scatter


def kernel(ligand_pos, ligand_v, edge_index, time_step, W_node, W_time,
           W_msg1, b_msg1, W_msg2, W_upd, W_coord, W_out):
    n = ligand_pos.shape[0]
    e = edge_index.shape[1]
    k = W_node.shape[0]
    h = W_node.shape[1]
    n_layers = W_msg1.shape[0]

    nblk_g = e // _NW // _BE
    nblk_s = e // _NS // _BE
    src3 = edge_index[0].astype(jnp.int32).reshape(_NW, nblk_g, _BE)
    dst3 = edge_index[1].astype(jnp.int32).reshape(_NW, nblk_g, _BE)
    dst3s = edge_index[1].astype(jnp.int32).reshape(_NS, 2, nblk_s // 2, _BE)
    v2 = ligand_v.astype(jnp.int32).reshape(n, 1)
    t2 = time_step.astype(jnp.int32).reshape(n, 1)
    half = h // 2
    freq = jnp.asarray(
        np.exp(np.arange(half, dtype=np.float32) * -(np.log(10000.0) / (half - 1)))
    ).reshape(1, half)
    zeros_nh = jnp.zeros((n, h), jnp.float32)

    gather = _make_gather(n, e, h)
    scatter = _make_scatter(n, e, h)

    hmat, x128 = _embed(v2, t2, ligand_pos, freq, W_node, W_time, n, h)

    for l in range(n_layers):
        w1a = W_msg1[l, :h, :]
        w1b = W_msg1[l, h:2 * h, :]
        w1d = W_msg1[l, 2 * h:2 * h + 1, :]
        bias = b_msg1[l].reshape(1, h)
        wc = W_coord[l, :, 0].reshape(1, h)

        t1h, t2h = _proj(hmat, w1a, w1b, n, h)
        g, gxr = gather(t1h, t2h, x128, src3, dst3)
        mh, mx = _edge(g, gxr, w1d, bias, W_msg2[l], wc, e, h)
        agg, dxs = scatter(mh, mx, dst3s, zeros_nh)
        hmat, x128 = _update(hmat, x128, agg, dxs, W_upd[l], n, h)

    return _final(hmat, x128, ligand_pos, W_out, n, h, k)
